# baseline (speedup 1.0000x reference)
import jax
import jax.numpy as jnp
from jax.experimental import pallas as pl
from jax.experimental.pallas import tpu as pltpu

_TINY = 1.1754943508222875e-38


def _make_stream_kernel(V, C, K, G):
    rows = 2 * K
    NBUF = 4

    def _stream(ids_smem, unif_smem, t_hbm, d_hbm, g_hbm, besti_ref, acc_ref,
                t_vmem, d_vmem, g_vmem, sem):
        i = pl.program_id(0)
        slot = jax.lax.rem(i, NBUF)

        def copies(step, s):
            out = []
            for half in range(2):
                b = 2 * step + half
                rr = half * K
                out.append(pltpu.make_async_copy(
                    t_hbm.at[b, 0:K, :], t_vmem.at[s, rr : rr + K, :],
                    sem.at[s, half]))
                out.append(pltpu.make_async_copy(
                    d_hbm.at[b, :, :], d_vmem.at[s, rr : rr + K, :],
                    sem.at[s, 2 + half]))
                out.append(pltpu.make_async_copy(
                    g_hbm.at[b, :, :], g_vmem.at[s, rr : rr + K, :],
                    sem.at[s, 4 + half]))
            return out

        @pl.when(i == 0)
        def _prologue():
            for cp in copies(0, 0) + copies(1, 1) + copies(2, 2):
                cp.start()

        @pl.when(i + 3 < G)
        def _prefetch():
            for cp in copies(i + 3, jax.lax.rem(i + 3, NBUF)):
                cp.start()

        for cp in copies(i, slot):
            cp.wait()

        best_v = jnp.full((rows, 1), -jnp.inf, jnp.float32)
        best_i = jnp.zeros((rows, 1), jnp.int32)
        for c in range(0, V, C):
            cc = min(C, V - c)
            t8 = t_vmem[slot, :, c : c + cc]
            d8 = d_vmem[slot, :, c : c + cc]
            g8 = g_vmem[slot, :, c : c + cc]
            score = jnp.log(jnp.maximum(t8 - d8, _TINY)) + g8
            m = jnp.max(score, axis=1, keepdims=True)
            lane = jax.lax.broadcasted_iota(jnp.int32, (rows, cc), 1)
            loc = jnp.min(jnp.where(score == m, lane, V), axis=1,
                          keepdims=True)
            upd = m > best_v
            best_v = jnp.where(upd, m, best_v)
            best_i = jnp.where(upd, c + loc, best_i)
        besti_ref[0] = best_i

        lane128 = jax.lax.broadcasted_iota(jnp.int32, (1, 128), 1)
        subl = jax.lax.broadcasted_iota(jnp.int32, (rows, 1), 0)
        acc = jnp.zeros((rows, 1), jnp.int32)
        for r in range(rows):
            b, kk = divmod(r, K)
            tid_s = ids_smem[0, b, kk]
            grp = pl.multiple_of((tid_s // 128) * 128, 128)
            tv = t_vmem[slot, r : r + 1, pl.ds(grp, 128)]
            dv = d_vmem[slot, r : r + 1, pl.ds(grp, 128)]
            msk = lane128 == (tid_s - grp)
            sel_t = jnp.sum(jnp.where(msk, tv, 0.0), axis=1, keepdims=True)
            sel_d = jnp.sum(jnp.where(msk, dv, 0.0), axis=1, keepdims=True)
            a = jnp.where(
                unif_smem[0, b, kk] < jnp.minimum(sel_t / sel_d, 1.0), 1, 0
            ).astype(jnp.int32)
            acc = jnp.where(subl == r, a, acc)
        acc_ref[0] = acc

    return _stream


def _epilogue(ids_ref, bonus_ref, besti_ref, acc_ref, out_ref):
    b, k = ids_ref.shape
    kidx = jax.lax.broadcasted_iota(jnp.int32, (b, k), 1)
    limits = jnp.min(jnp.where(acc_ref[...] == 0, kidx, k), axis=1,
                     keepdims=True)
    out_k = jnp.where(kidx < limits, ids_ref[...], -1)
    bonus_col = jnp.where(out_k[:, k - 1 : k] != -1, bonus_ref[...], -1)
    out_k = jnp.where(kidx == limits, besti_ref[...], out_k)
    out_ref[:, :k] = out_k
    out_ref[:, k:] = bonus_col


@jax.jit
def kernel(target_with_bonus_probs, bonus_token_ids, draft_probs,
           draft_token_ids, uniform_rand, gumbel_noise):
    B, K, V = draft_probs.shape
    C = 1024
    G = B // 2
    rows = 2 * K
    ids3 = draft_token_ids.reshape(G, 2, K)
    unif3 = uniform_rand.reshape(G, 2, K)
    besti, acc = pl.pallas_call(
        _make_stream_kernel(V, C, K, G),
        grid=(G,),
        in_specs=[
            pl.BlockSpec((1, 2, K), lambda i: (i, 0, 0),
                         memory_space=pltpu.SMEM),
            pl.BlockSpec((1, 2, K), lambda i: (i, 0, 0),
                         memory_space=pltpu.SMEM),
            pl.BlockSpec(memory_space=pl.ANY),
            pl.BlockSpec(memory_space=pl.ANY),
            pl.BlockSpec(memory_space=pl.ANY),
        ],
        out_specs=[
            pl.BlockSpec((1, rows, 1), lambda i: (i, 0, 0)),
            pl.BlockSpec((1, rows, 1), lambda i: (i, 0, 0)),
        ],
        out_shape=[
            jax.ShapeDtypeStruct((G, rows, 1), jnp.int32),
            jax.ShapeDtypeStruct((G, rows, 1), jnp.int32),
        ],
        scratch_shapes=[
            pltpu.VMEM((4, rows, V), jnp.float32),
            pltpu.VMEM((4, rows, V), jnp.float32),
            pltpu.VMEM((4, rows, V), jnp.float32),
            pltpu.SemaphoreType.DMA((4, 6)),
        ],
        compiler_params=pltpu.CompilerParams(
            dimension_semantics=("arbitrary",),
        ),
    )(ids3, unif3, target_with_bonus_probs, draft_probs, gumbel_noise)

    out = pl.pallas_call(
        _epilogue,
        out_shape=jax.ShapeDtypeStruct((B, K + 1), jnp.int32),
    )(draft_token_ids, bonus_token_ids, besti.reshape(B, K),
      acc.reshape(B, K))
    return out
